# baseline (speedup 1.0000x reference)
import functools

import jax
import jax.numpy as jnp
from jax import lax
from jax.experimental import pallas as pl
from jax.experimental.pallas import tpu as pltpu
from jax.experimental.pallas import tpu_sc as plsc

_LANES = 16
_UNROLL = 16


def _xlane_take(x, perm):
    dnums = lax.GatherDimensionNumbers(
        offset_dims=(), collapsed_slice_dims=(0,), start_index_map=(0,))
    return lax.gather(x, perm[:, None], dnums, slice_sizes=(1,),
                      mode=lax.GatherScatterMode.PROMISE_IN_BOUNDS)


def _merge(m_a, i_a, m_b, i_b):
    take_b = (m_b > m_a) | ((m_b == m_a) & (i_b < i_a))
    return jnp.where(take_b, m_b, m_a), jnp.where(take_b, i_b, i_a)


def _make_wta(rows, cols):
    info = plsc.get_sparse_core_info()
    ncores, nsub = info.num_cores, info.num_subcores
    nworkers = ncores * nsub
    assert rows % nworkers == 0
    rows_per = rows // nworkers
    assert rows_per <= _LANES
    assert cols % (_LANES * _UNROLL) == 0
    steps = cols // (_LANES * _UNROLL)

    mesh = plsc.VectorSubcoreMesh(core_axis_name="c", subcore_axis_name="s")

    @functools.partial(
        pl.kernel,
        out_type=jax.ShapeDtypeStruct((rows * cols,), jnp.float32),
        mesh=mesh,
        scratch_types=[
            pltpu.VMEM((cols,), jnp.float32),
            pltpu.VMEM((cols,), jnp.float32),
            pltpu.VMEM((cols,), jnp.float32),
            pltpu.VMEM((_LANES,), jnp.float32),
            pltpu.VMEM((_LANES,), jnp.int32),
            pltpu.SemaphoreType.DMA,
            pltpu.SemaphoreType.DMA,
            pltpu.SemaphoreType.DMA,
            pltpu.SemaphoreType.DMA,
        ],
    )
    def wta(x_hbm, out_hbm, in0, in1, zrow, ones_v, idxs_v, sem0, sem1, semz, sems):
        wid = lax.axis_index("s") * ncores + lax.axis_index("c")
        row0 = wid * rows_per

        bufs = (in0, in1)
        sems_in = (sem0, sem1)
        in_copies = [None] * rows_per
        for r in range(min(2, rows_per)):
            in_copies[r] = pltpu.async_copy(
                x_hbm.at[pl.ds((row0 + r) * cols, cols)], bufs[r % 2], sems_in[r % 2])

        zero_vec = jnp.zeros((_LANES,), jnp.float32)
        def zfill(i, _):
            for u in range(8):
                zrow[pl.ds(i * (8 * _LANES) + u * _LANES, _LANES)] = zero_vec
            return 0
        lax.fori_loop(0, cols // (8 * _LANES), zfill, 0)
        ones_v[...] = jnp.ones((_LANES,), jnp.float32)
        z_copies = [
            pltpu.async_copy(zrow, out_hbm.at[pl.ds((row0 + r) * cols, cols)], semz)
            for r in range(rows_per)
        ]

        lane = lax.iota(jnp.int32, _LANES)
        winners = jnp.zeros((_LANES,), jnp.int32)
        neg_inf = jnp.full((_LANES,), -jnp.inf, jnp.float32)
        zero_i = jnp.zeros((_LANES,), jnp.int32)

        for r in range(rows_per):
            in_copies[r].wait()
            buf = bufs[r % 2]

            def step(j, carry):
                ms, tags = carry
                base = j * (_UNROLL * _LANES)
                new_ms, new_tags = [], []
                for u in range(_UNROLL):
                    v = buf[pl.ds(base + u * _LANES, _LANES)]
                    gt = v > ms[u]
                    new_ms.append(jnp.where(gt, v, ms[u]))
                    new_tags.append(jnp.where(gt, j, tags[u]))
                return tuple(new_ms), tuple(new_tags)

            init = ((neg_inf,) * _UNROLL, (zero_i,) * _UNROLL)
            ms, tags = lax.fori_loop(0, steps, step, init)

            pairs = [
                (ms[u], tags[u] * (_UNROLL * _LANES) + (u * _LANES) + lane)
                for u in range(_UNROLL)
            ]
            while len(pairs) > 1:
                nxt = []
                for p in range(0, len(pairs), 2):
                    nxt.append(_merge(*pairs[p], *pairs[p + 1]))
                pairs = nxt
            m, idx = pairs[0]

            for k in (8, 4, 2, 1):
                perm = lane ^ k
                m2 = _xlane_take(m, perm)
                i2 = _xlane_take(idx, perm)
                m, idx = _merge(m, idx, m2, i2)
            flat_win = idx + (row0 + r) * cols
            if r == 0:
                winners = flat_win
            else:
                winners = jnp.where(lane == r, flat_win, winners)

            if r + 2 < rows_per:
                in_copies[r + 2] = pltpu.async_copy(
                    x_hbm.at[pl.ds((row0 + r + 2) * cols, cols)],
                    bufs[r % 2], sems_in[r % 2])

        idxs_v[...] = winners
        for zc in z_copies:
            zc.wait()
        pltpu.async_copy(ones_v, out_hbm.at[idxs_v], sems).wait()

    return wta


def kernel(tensor):
    rows, cols = tensor.shape[0], int(jnp.size(tensor)) // tensor.shape[0]
    flat = tensor.reshape(rows * cols)
    out = _make_wta(rows, cols)(flat)
    return out.reshape(tensor.shape)


# speedup vs baseline: 1.9619x; 1.9619x over previous
import functools

import jax
import jax.numpy as jnp
from jax import lax
from jax.experimental import pallas as pl
from jax.experimental.pallas import tpu as pltpu
from jax.experimental.pallas import tpu_sc as plsc

_LANES = 16
_UNROLL = 16
_TR, _TC = 8, 128


def _xlane_take(x, perm):
    dnums = lax.GatherDimensionNumbers(
        offset_dims=(), collapsed_slice_dims=(0,), start_index_map=(0,))
    return lax.gather(x, perm[:, None], dnums, slice_sizes=(1,),
                      mode=lax.GatherScatterMode.PROMISE_IN_BOUNDS)


def _merge(m_a, i_a, m_b, i_b):
    take_b = (m_b > m_a) | ((m_b == m_a) & (i_b < i_a))
    return jnp.where(take_b, m_b, m_a), jnp.where(take_b, i_b, i_a)


def _make_wta(rows, cols):
    info = plsc.get_sparse_core_info()
    ncores, nsub = info.num_cores, info.num_subcores
    nworkers = ncores * nsub
    assert rows % nworkers == 0 and rows % _TR == 0 and cols % _TC == 0
    rows_per = rows // nworkers
    assert rows_per <= _LANES
    assert cols % (_LANES * _UNROLL) == 0
    steps = cols // (_LANES * _UNROLL)
    segs = cols // _TC
    seg_per_step = (_LANES * _UNROLL) // _TC

    mesh = plsc.VectorSubcoreMesh(core_axis_name="c", subcore_axis_name="s")

    @functools.partial(
        pl.kernel,
        out_type=jax.ShapeDtypeStruct((rows * cols,), jnp.float32),
        mesh=mesh,
        scratch_types=[
            pltpu.VMEM((segs, _TC), jnp.float32),
            pltpu.VMEM((segs, _TC), jnp.float32),
            pltpu.VMEM((cols,), jnp.float32),
            pltpu.VMEM((_LANES,), jnp.float32),
            pltpu.VMEM((_LANES,), jnp.int32),
            pltpu.SemaphoreType.DMA,
            pltpu.SemaphoreType.DMA,
            pltpu.SemaphoreType.DMA,
            pltpu.SemaphoreType.DMA,
        ],
    )
    def wta(x_hbm, out_hbm, in0, in1, zblk, ones_v, idxs_v, sem0, sem1, semz, sems):
        wid = lax.axis_index("c") * nsub + lax.axis_index("s")
        row0 = wid * rows_per

        bufs = (in0, in1)
        sems_in = (sem0, sem1)

        def start_in(r, buf, sem):
            rr = row0 + r
            return pltpu.async_copy(
                x_hbm.at[rr // _TR, :, rr % _TR, :], buf, sem)

        in_copies = [None] * rows_per
        for r in range(min(2, rows_per)):
            in_copies[r] = start_in(r, bufs[r % 2], sems_in[r % 2])

        zero_vec = jnp.zeros((_LANES,), jnp.float32)
        def zfill(i, _):
            for u in range(8):
                zblk[pl.ds(i * (8 * _LANES) + u * _LANES, _LANES)] = zero_vec
            return 0
        lax.fori_loop(0, cols // (8 * _LANES), zfill, 0)
        ones_v[...] = jnp.ones((_LANES,), jnp.float32)
        z_copies = [
            pltpu.async_copy(zblk, out_hbm.at[pl.ds((row0 + r) * cols, cols)], semz)
            for r in range(rows_per)
        ]

        lane = lax.iota(jnp.int32, _LANES)
        winners = jnp.zeros((_LANES,), jnp.int32)
        neg_inf = jnp.full((_LANES,), -jnp.inf, jnp.float32)
        zero_i = jnp.zeros((_LANES,), jnp.int32)

        for r in range(rows_per):
            in_copies[r].wait()
            buf = bufs[r % 2]

            def step(j, carry):
                ms, tags = carry
                new_ms, new_tags = [], []
                for u in range(_UNROLL):
                    v = buf[j * seg_per_step + u // (_TC // _LANES),
                            pl.ds((u % (_TC // _LANES)) * _LANES, _LANES)]
                    gt = v > ms[u]
                    new_ms.append(jnp.where(gt, v, ms[u]))
                    new_tags.append(jnp.where(gt, j, tags[u]))
                return tuple(new_ms), tuple(new_tags)

            init = ((neg_inf,) * _UNROLL, (zero_i,) * _UNROLL)
            ms, tags = lax.fori_loop(0, steps, step, init)

            pairs = [
                (ms[u], tags[u] * (_UNROLL * _LANES) + (u * _LANES) + lane)
                for u in range(_UNROLL)
            ]
            while len(pairs) > 1:
                nxt = []
                for p in range(0, len(pairs), 2):
                    nxt.append(_merge(*pairs[p], *pairs[p + 1]))
                pairs = nxt
            m, idx = pairs[0]

            for k in (8, 4, 2, 1):
                perm = lane ^ k
                m2 = _xlane_take(m, perm)
                i2 = _xlane_take(idx, perm)
                m, idx = _merge(m, idx, m2, i2)

            rr = row0 + r
            flat_win = ((rr // _TR) * (_TR * cols) + (rr % _TR) * _TC
                        + ((idx >> 7) << 10) + (idx & (_TC - 1)))
            if r == 0:
                winners = flat_win
            else:
                winners = jnp.where(lane == r, flat_win, winners)

            if r + 2 < rows_per:
                in_copies[r + 2] = start_in(r + 2, bufs[r % 2], sems_in[r % 2])

        idxs_v[...] = winners
        for zc in z_copies:
            zc.wait()
        plsc.subcore_barrier()
        pltpu.async_copy(ones_v, out_hbm.at[idxs_v], sems).wait()

    return wta


def kernel(tensor):
    rows, cols = tensor.shape
    x4 = tensor.reshape(rows // _TR, _TR, cols // _TC, _TC).transpose(0, 2, 1, 3)
    flat = _make_wta(rows, cols)(x4)
    out = (flat.reshape(rows // _TR, cols // _TC, _TR, _TC)
           .transpose(0, 2, 1, 3).reshape(rows, cols))
    return out
